# Initial kernel scaffold; baseline (speedup 1.0000x reference)
import jax
import jax.numpy as jnp
from jax import lax
from jax.experimental import pallas as pl
from jax.experimental.pallas import tpu as pltpu
from jax.experimental.pallas import tpu_sc as plsc

_NC = 2
_NS = 16
_NW = _NC * _NS
_D = 64
_B = 4096 * 200
_PER_W = _B // _NW
_CHUNK = 128
_NCHUNK = _PER_W // _CHUNK


def _body(tok_hbm, tab_hbm, out_hbm, idx_v, rows_v, sem):
    wid = lax.axis_index("s") * _NC + lax.axis_index("c")
    pltpu.sync_copy(tok_hbm.at[wid], idx_v)

    def chunk(j, carry):
        pltpu.async_copy(tab_hbm.at[idx_v.at[j]], rows_v, sem).wait()
        pltpu.sync_copy(rows_v, out_hbm.at[wid, j])
        return carry

    lax.fori_loop(0, _NCHUNK, chunk, 0)


@jax.jit
def kernel(token_ids, embeddings):
    S, T = token_ids.shape
    tok = token_ids.reshape(_NW, _NCHUNK, _CHUNK).astype(jnp.int32)
    out = pl.kernel(
        _body,
        out_type=jax.ShapeDtypeStruct((_NW, _NCHUNK, _CHUNK, _D), jnp.float32),
        mesh=plsc.VectorSubcoreMesh(core_axis_name="c", subcore_axis_name="s"),
        scratch_types=[
            pltpu.VMEM((_NCHUNK, _CHUNK), jnp.int32),
            pltpu.VMEM((_CHUNK, _D), jnp.float32),
            pltpu.SemaphoreType.DMA,
        ],
    )(tok, embeddings)
    return out.reshape(S, T, _D)


# speedup vs baseline: 3.5404x; 3.5404x over previous
import jax
import jax.numpy as jnp
from jax import lax
from jax.experimental import pallas as pl
from jax.experimental.pallas import tpu as pltpu
from jax.experimental.pallas import tpu_sc as plsc

_NC = 2
_NS = 16
_NW = _NC * _NS
_D = 64
_B = 4096 * 200
_PER_W = _B // _NW
_CHUNK = 128
_NCHUNK = _PER_W // _CHUNK


def _body(tok_hbm, tab_hbm, out_hbm, idx_v, rows_v, sem):
    wid = lax.axis_index("s") * _NC + lax.axis_index("c")
    pltpu.sync_copy(tok_hbm.at[wid], idx_v)

    def chunk(j, carry):
        pltpu.async_copy(tab_hbm.at[idx_v.at[j]], rows_v, sem).wait()
        pltpu.sync_copy(rows_v, out_hbm.at[wid, j])
        return carry

    lax.fori_loop(0, _NCHUNK, chunk, 0)


@jax.jit
def kernel(token_ids, embeddings):
    S, T = token_ids.shape
    tok = token_ids.reshape(_NW, _NCHUNK, _CHUNK).astype(jnp.int32)
    out = pl.kernel(
        _body,
        out_type=jax.ShapeDtypeStruct((_NW, _NCHUNK, _CHUNK, _D), jnp.float32),
        mesh=plsc.VectorSubcoreMesh(core_axis_name="c", subcore_axis_name="s"),
        compiler_params=pltpu.CompilerParams(use_tc_tiling_on_sc=False),
        scratch_types=[
            pltpu.VMEM((_NCHUNK, _CHUNK), jnp.int32),
            pltpu.VMEM((_CHUNK, _D), jnp.float32),
            pltpu.SemaphoreType.DMA,
        ],
    )(tok, embeddings)
    return out.reshape(S, T, _D)


# speedup vs baseline: 4.2594x; 1.2031x over previous
import jax
import jax.numpy as jnp
from jax import lax
from jax.experimental import pallas as pl
from jax.experimental.pallas import tpu as pltpu
from jax.experimental.pallas import tpu_sc as plsc

_NC = 2
_NS = 16
_NW = _NC * _NS
_D = 64
_B = 4096 * 200
_PER_W = _B // _NW
_CHUNK = 128
_NCHUNK = _PER_W // _CHUNK
_NBUF = 8


def _body(tok_hbm, tab_hbm, out_hbm, idx_v, rows_v, gsem, ssem):
    wid = lax.axis_index("s") * _NC + lax.axis_index("c")
    pltpu.sync_copy(tok_hbm.at[wid], idx_v)

    def gather(j, b):
        return pltpu.make_async_copy(tab_hbm.at[idx_v.at[j]], rows_v.at[b], gsem.at[b])

    def store(j, b):
        return pltpu.make_async_copy(rows_v.at[b], out_hbm.at[wid, j], ssem.at[b])

    for b in range(_NBUF):
        gather(b, b).start()

    @pl.loop(0, _NCHUNK - _NBUF, step=_NBUF)
    def _(g):
        for b in range(_NBUF):
            gather(g + b, b).wait()
            store(g + b, b).start()
        for b in range(_NBUF):
            store(g + b, b).wait()
            gather(g + _NBUF + b, b).start()

    for b in range(_NBUF):
        gather(_NCHUNK - _NBUF + b, b).wait()
        store(_NCHUNK - _NBUF + b, b).start()
    for b in range(_NBUF):
        store(_NCHUNK - _NBUF + b, b).wait()


@jax.jit
def kernel(token_ids, embeddings):
    S, T = token_ids.shape
    tok = token_ids.reshape(_NW, _NCHUNK, _CHUNK).astype(jnp.int32)
    out = pl.kernel(
        _body,
        out_type=jax.ShapeDtypeStruct((_NW, _NCHUNK, _CHUNK, _D), jnp.float32),
        mesh=plsc.VectorSubcoreMesh(core_axis_name="c", subcore_axis_name="s"),
        compiler_params=pltpu.CompilerParams(use_tc_tiling_on_sc=False),
        scratch_types=[
            pltpu.VMEM((_NCHUNK, _CHUNK), jnp.int32),
            pltpu.VMEM((_NBUF, _CHUNK, _D), jnp.float32),
            pltpu.SemaphoreType.DMA((_NBUF,)),
            pltpu.SemaphoreType.DMA((_NBUF,)),
        ],
    )(tok, embeddings)
    return out.reshape(S, T, _D)
